# baseline (speedup 1.0000x reference)
import jax
import jax.numpy as jnp
from jax.experimental import pallas as pl
from jax.experimental.pallas import tpu as pltpu

_BLK = 16384
_K = 1024


def _round_bf16(x):
    u = jax.lax.bitcast_convert_type(x, jnp.uint32)
    u = u + jnp.uint32(0x7FFF) + ((u >> 16) & jnp.uint32(1))
    u = u & jnp.uint32(0xFFFF0000)
    return jax.lax.bitcast_convert_type(u, jnp.float32)


def _vq_body(ct_ref, ctb_ref, zt_ref, idx_ref, qt_ref, z2_ref, zxb_ref, zyb_ref):
    zx = zt_ref[0, :]
    zy = zt_ref[1, :]
    z2_ref[...] = zx * zx + zy * zy
    zxb_ref[...] = _round_bf16(zx)
    zyb_ref[...] = _round_bf16(zy)

    def body(k, carry):
        bd, bi, bx, by = carry
        cx = ct_ref[0, k]
        cy = ct_ref[1, k]
        cxb = ctb_ref[0, k]
        cyb = ctb_ref[1, k]
        dot = zxb_ref[...] * cxb + zyb_ref[...] * cyb
        c2 = cx * cx + cy * cy
        u = z2_ref[...] + c2
        d2 = jnp.maximum(u - (dot + dot), 0.0)
        m = d2 < bd
        bd = jnp.where(m, d2, bd)
        bi = jnp.where(m, k, bi)
        bx = jnp.where(m, cx, bx)
        by = jnp.where(m, cy, by)
        return bd, bi, bx, by

    init = (
        jnp.full((_BLK,), jnp.inf, jnp.float32),
        jnp.zeros((_BLK,), jnp.int32),
        jnp.zeros((_BLK,), jnp.float32),
        jnp.zeros((_BLK,), jnp.float32),
    )
    _, bi, bx, by = jax.lax.fori_loop(0, _K, body, init, unroll=32)
    idx_ref[...] = bi
    qt_ref[0, :] = bx
    qt_ref[1, :] = by


def kernel(z, codebook):
    n = z.shape[0]
    zt = z.T
    ct = codebook.T
    ctb = _round_bf16(ct)
    idx, qt = pl.pallas_call(
        _vq_body,
        grid=(n // _BLK,),
        in_specs=[
            pl.BlockSpec(memory_space=pltpu.SMEM),
            pl.BlockSpec(memory_space=pltpu.SMEM),
            pl.BlockSpec((2, _BLK), lambda i: (0, i)),
        ],
        out_specs=[
            pl.BlockSpec((_BLK,), lambda i: (i,)),
            pl.BlockSpec((2, _BLK), lambda i: (0, i)),
        ],
        out_shape=[
            jax.ShapeDtypeStruct((n,), jnp.int32),
            jax.ShapeDtypeStruct((2, n), jnp.float32),
        ],
        scratch_shapes=[
            pltpu.VMEM((_BLK,), jnp.float32),
            pltpu.VMEM((_BLK,), jnp.float32),
            pltpu.VMEM((_BLK,), jnp.float32),
        ],
    )(ct, ctb, zt)
    q = qt.T
    return (q, idx, q)


# speedup vs baseline: 1.1908x; 1.1908x over previous
import jax
import jax.numpy as jnp
from jax.experimental import pallas as pl
from jax.experimental.pallas import tpu as pltpu

_BLK = 8192
_K = 1024


def _round_bf16(x):
    u = jax.lax.bitcast_convert_type(x, jnp.uint32)
    u = u + jnp.uint32(0x7FFF) + ((u >> 16) & jnp.uint32(1))
    u = u & jnp.uint32(0xFFFF0000)
    return jax.lax.bitcast_convert_type(u, jnp.float32)


def _vq_body(ct_ref, ctb_ref, zt_ref, idx_ref, qt_ref, z2_ref, zxb_ref, zyb_ref):
    zx = zt_ref[0, :]
    zy = zt_ref[1, :]
    z2_ref[...] = zx * zx + zy * zy
    zxb_ref[...] = _round_bf16(zx)
    zyb_ref[...] = _round_bf16(zy)

    def body(k, carry):
        bd, bi, bx, by = carry
        cx = ct_ref[0, k]
        cy = ct_ref[1, k]
        cxb = ctb_ref[0, k]
        cyb = ctb_ref[1, k]
        dot = zxb_ref[...] * cxb + zyb_ref[...] * cyb
        c2 = cx * cx + cy * cy
        u = z2_ref[...] + c2
        d2 = jnp.maximum(u - (dot + dot), 0.0)
        m = d2 < bd
        bd = jnp.where(m, d2, bd)
        bi = jnp.where(m, k, bi)
        bx = jnp.where(m, cx, bx)
        by = jnp.where(m, cy, by)
        return bd, bi, bx, by

    init = (
        jnp.full((_BLK,), jnp.inf, jnp.float32),
        jnp.zeros((_BLK,), jnp.int32),
        jnp.zeros((_BLK,), jnp.float32),
        jnp.zeros((_BLK,), jnp.float32),
    )
    _, bi, bx, by = jax.lax.fori_loop(0, _K, body, init, unroll=64)
    idx_ref[...] = bi
    qt_ref[0, :] = bx
    qt_ref[1, :] = by


def kernel(z, codebook):
    n = z.shape[0]
    zt = z.T
    ct = codebook.T
    ctb = _round_bf16(ct)
    idx, qt = pl.pallas_call(
        _vq_body,
        grid=(n // _BLK,),
        in_specs=[
            pl.BlockSpec(memory_space=pltpu.SMEM),
            pl.BlockSpec(memory_space=pltpu.SMEM),
            pl.BlockSpec((2, _BLK), lambda i: (0, i)),
        ],
        out_specs=[
            pl.BlockSpec((_BLK,), lambda i: (i,)),
            pl.BlockSpec((2, _BLK), lambda i: (0, i)),
        ],
        out_shape=[
            jax.ShapeDtypeStruct((n,), jnp.int32),
            jax.ShapeDtypeStruct((2, n), jnp.float32),
        ],
        scratch_shapes=[
            pltpu.VMEM((_BLK,), jnp.float32),
            pltpu.VMEM((_BLK,), jnp.float32),
            pltpu.VMEM((_BLK,), jnp.float32),
        ],
    )(ct, ctb, zt)
    q = qt.T
    return (q, idx, q)


# speedup vs baseline: 1.1985x; 1.0064x over previous
import jax
import jax.numpy as jnp
from jax.experimental import pallas as pl
from jax.experimental.pallas import tpu as pltpu

_BLK = 8192
_K = 1024


def _round_bf16(x):
    u = jax.lax.bitcast_convert_type(x, jnp.uint32)
    u = u + jnp.uint32(0x7FFF) + ((u >> 16) & jnp.uint32(1))
    u = u & jnp.uint32(0xFFFF0000)
    return jax.lax.bitcast_convert_type(u, jnp.float32)


def _vq_body(ct_ref, ctb_ref, zt_ref, idx_ref, qt_ref, z2_ref, zxb_ref, zyb_ref):
    zx = zt_ref[0, :]
    zy = zt_ref[1, :]
    z2_ref[...] = zx * zx + zy * zy
    zxb_ref[...] = _round_bf16(zx)
    zyb_ref[...] = _round_bf16(zy)

    def body(k, carry):
        bd, bi, bx, by = carry
        cx = ct_ref[0, k]
        cy = ct_ref[1, k]
        cxb = ctb_ref[0, k]
        cyb = ctb_ref[1, k]
        dot = zxb_ref[...] * cxb + zyb_ref[...] * cyb
        c2 = cx * cx + cy * cy
        u = z2_ref[...] + c2
        d2 = jnp.maximum(u - (dot + dot), 0.0)
        m = d2 < bd
        bd = jnp.where(m, d2, bd)
        bi = jnp.where(m, k, bi)
        bx = jnp.where(m, cx, bx)
        by = jnp.where(m, cy, by)
        return bd, bi, bx, by

    init = (
        jnp.full((_BLK,), jnp.inf, jnp.float32),
        jnp.zeros((_BLK,), jnp.int32),
        jnp.zeros((_BLK,), jnp.float32),
        jnp.zeros((_BLK,), jnp.float32),
    )
    _, bi, bx, by = jax.lax.fori_loop(0, _K, body, init, unroll=128)
    idx_ref[...] = bi
    qt_ref[0, :] = bx
    qt_ref[1, :] = by


def kernel(z, codebook):
    n = z.shape[0]
    zt = z.T
    ct = codebook.T
    ctb = _round_bf16(ct)
    idx, qt = pl.pallas_call(
        _vq_body,
        grid=(n // _BLK,),
        in_specs=[
            pl.BlockSpec(memory_space=pltpu.SMEM),
            pl.BlockSpec(memory_space=pltpu.SMEM),
            pl.BlockSpec((2, _BLK), lambda i: (0, i)),
        ],
        out_specs=[
            pl.BlockSpec((_BLK,), lambda i: (i,)),
            pl.BlockSpec((2, _BLK), lambda i: (0, i)),
        ],
        out_shape=[
            jax.ShapeDtypeStruct((n,), jnp.int32),
            jax.ShapeDtypeStruct((2, n), jnp.float32),
        ],
        scratch_shapes=[
            pltpu.VMEM((_BLK,), jnp.float32),
            pltpu.VMEM((_BLK,), jnp.float32),
            pltpu.VMEM((_BLK,), jnp.float32),
        ],
    )(ct, ctb, zt)
    q = qt.T
    return (q, idx, q)


# speedup vs baseline: 1.2022x; 1.0031x over previous
import jax
import jax.numpy as jnp
from jax.experimental import pallas as pl
from jax.experimental.pallas import tpu as pltpu

_BLK = 8192
_K = 1024


def _round_bf16(x):
    u = jax.lax.bitcast_convert_type(x, jnp.uint32)
    u = u + jnp.uint32(0x7FFF) + ((u >> 16) & jnp.uint32(1))
    u = u & jnp.uint32(0xFFFF0000)
    return jax.lax.bitcast_convert_type(u, jnp.float32)


def _vq_body(ct_ref, ctb_ref, zt_ref, idx_ref, qt_ref, z2_ref, zxb_ref, zyb_ref):
    zx = zt_ref[0, :]
    zy = zt_ref[1, :]
    z2_ref[...] = zx * zx + zy * zy
    zxb_ref[...] = _round_bf16(zx)
    zyb_ref[...] = _round_bf16(zy)

    def body(k, carry):
        bd, bi, bx, by = carry
        cx = ct_ref[0, k]
        cy = ct_ref[1, k]
        cxb = ctb_ref[0, k]
        cyb = ctb_ref[1, k]
        dot = zxb_ref[...] * cxb + zyb_ref[...] * cyb
        c2 = cx * cx + cy * cy
        u = z2_ref[...] + c2
        d2 = jnp.maximum(u - (dot + dot), 0.0)
        m = d2 < bd
        bd = jnp.where(m, d2, bd)
        bi = jnp.where(m, k, bi)
        bx = jnp.where(m, cx, bx)
        by = jnp.where(m, cy, by)
        return bd, bi, bx, by

    init = (
        jnp.full((_BLK,), jnp.inf, jnp.float32),
        jnp.zeros((_BLK,), jnp.int32),
        jnp.zeros((_BLK,), jnp.float32),
        jnp.zeros((_BLK,), jnp.float32),
    )
    _, bi, bx, by = jax.lax.fori_loop(0, _K, body, init, unroll=256)
    idx_ref[...] = bi
    qt_ref[0, :] = bx
    qt_ref[1, :] = by


def kernel(z, codebook):
    n = z.shape[0]
    zt = z.T
    ct = codebook.T
    ctb = _round_bf16(ct)
    idx, qt = pl.pallas_call(
        _vq_body,
        grid=(n // _BLK,),
        in_specs=[
            pl.BlockSpec(memory_space=pltpu.SMEM),
            pl.BlockSpec(memory_space=pltpu.SMEM),
            pl.BlockSpec((2, _BLK), lambda i: (0, i)),
        ],
        out_specs=[
            pl.BlockSpec((_BLK,), lambda i: (i,)),
            pl.BlockSpec((2, _BLK), lambda i: (0, i)),
        ],
        out_shape=[
            jax.ShapeDtypeStruct((n,), jnp.int32),
            jax.ShapeDtypeStruct((2, n), jnp.float32),
        ],
        scratch_shapes=[
            pltpu.VMEM((_BLK,), jnp.float32),
            pltpu.VMEM((_BLK,), jnp.float32),
            pltpu.VMEM((_BLK,), jnp.float32),
        ],
    )(ct, ctb, zt)
    q = qt.T
    return (q, idx, q)
